# speedup vs baseline: 6.3187x; 6.3187x over previous
import functools

import jax
import jax.numpy as jnp
from jax.experimental import pallas as pl
from jax.experimental.pallas import tpu as pltpu

N = 100000
K = 16
D = 128
T = 512
BLK = 2000


def _body(x_ref, w_ref, idx_ref, tab_ref, o_ref):
    idx = idx_ref[...]
    w = w_ref[...]
    cols = jax.lax.broadcasted_iota(jnp.int32, (BLK, T), 1)
    acc = jnp.zeros((BLK, T), jnp.float32)
    for k in range(K):
        hit = cols == idx[:, k][:, None]
        acc = acc + jnp.where(hit, w[:, k][:, None], 0.0)
    o_ref[...] = x_ref[...] + jnp.dot(
        acc, tab_ref[...], preferred_element_type=jnp.float32)


def kernel(x, degree_weights, degree_indices, degree_table):
    idx = degree_indices.astype(jnp.int32)
    grid = (N // BLK,)
    return pl.pallas_call(
        _body,
        grid=grid,
        in_specs=[
            pl.BlockSpec((BLK, D), lambda i: (i, 0)),
            pl.BlockSpec((BLK, K), lambda i: (i, 0)),
            pl.BlockSpec((BLK, K), lambda i: (i, 0)),
            pl.BlockSpec((T, D), lambda i: (0, 0)),
        ],
        out_specs=pl.BlockSpec((BLK, D), lambda i: (i, 0)),
        out_shape=jax.ShapeDtypeStruct((N, D), jnp.float32),
    )(x, degree_weights, idx, degree_table)
